# baseline (speedup 1.0000x reference)
import functools

import jax
import jax.numpy as jnp
from jax.experimental import pallas as pl
from jax.experimental.pallas import tpu as pltpu

N_NODES = 50000
IN_FEATS = 1433
HID = 128
OUT = 7


def _mm_scale_kernel(x_ref, ns_ref, w_ref, o_ref):
    x = x_ref[...] * ns_ref[...]
    o_ref[...] = jax.lax.dot(x, w_ref[...], preferred_element_type=jnp.float32)


def _scaled_matmul(x, norm_src, w, block_rows=256):
    n, k = x.shape
    _, m = w.shape
    grid = (pl.cdiv(n, block_rows),)
    return pl.pallas_call(
        _mm_scale_kernel,
        grid=grid,
        in_specs=[
            pl.BlockSpec((block_rows, k), lambda i: (i, 0)),
            pl.BlockSpec((block_rows, 1), lambda i: (i, 0)),
            pl.BlockSpec((k, m), lambda i: (0, 0)),
        ],
        out_specs=pl.BlockSpec((block_rows, m), lambda i: (i, 0)),
        out_shape=jax.ShapeDtypeStruct((n, m), jnp.float32),
    )(x, norm_src[:, None], w)


def kernel(features, edge_index, W1, b1, W2, b2):
    src = edge_index[0].astype(jnp.int32)
    dst = edge_index[1].astype(jnp.int32)

    deg_out = jnp.zeros((N_NODES,), jnp.float32).at[src].add(1.0)
    deg_in = jnp.zeros((N_NODES,), jnp.float32).at[dst].add(1.0)
    norm_src = jnp.where(deg_out > 0, jax.lax.rsqrt(deg_out), 0.0)
    norm_dst = jnp.where(deg_in > 0, jax.lax.rsqrt(deg_in), 0.0)

    h = _scaled_matmul(features, norm_src, W1)
    msg = jnp.take(h, src, axis=0)
    agg = jax.ops.segment_sum(msg, dst, num_segments=N_NODES)
    x1 = jax.nn.relu(agg * norm_dst[:, None] + b1)

    w2p = jnp.zeros((HID, 128), jnp.float32).at[:, :OUT].set(W2)
    h2 = _scaled_matmul(x1, norm_src, w2p)
    msg2 = jnp.take(h2, src, axis=0)
    agg2 = jax.ops.segment_sum(msg2, dst, num_segments=N_NODES)
    out = agg2[:, :OUT] * norm_dst[:, None] + b2
    return out


# speedup vs baseline: 3.4944x; 3.4944x over previous
import functools

import jax
import jax.numpy as jnp
from jax import lax
from jax.experimental import pallas as pl
from jax.experimental.pallas import tpu as pltpu
from jax.experimental.pallas import tpu_sc as plsc

N_NODES = 50000
N_EDGES = 800000
IN_FEATS = 1433
HID = 128
OUT = 7

NPAD = 50176
NW = 32
EPW = N_EDGES // NW
SEG = 5000
BATCH = 128
NBATCH = N_EDGES // BATCH
STRIPE = NPAD // 16

_sc_mesh = plsc.VectorSubcoreMesh(core_axis_name="c", subcore_axis_name="s")


@functools.partial(
    pl.kernel,
    out_type=[
        jax.ShapeDtypeStruct((NW, NPAD), jnp.float32),
        jax.ShapeDtypeStruct((NW, NPAD), jnp.float32),
    ],
    mesh=_sc_mesh,
    scratch_types=[
        pltpu.VMEM((NPAD,), jnp.float32),
        pltpu.VMEM((NPAD,), jnp.float32),
        pltpu.VMEM((SEG,), jnp.int32),
    ],
    compiler_params=pltpu.CompilerParams(needs_layout_passes=False),
)
def _deg_kernel(src_hbm, dst_hbm, dego_hbm, degi_hbm, acc_o, acc_i, seg_v):
    cid = lax.axis_index("c")
    sid = lax.axis_index("s")
    wid = sid * 2 + cid

    zeros = jnp.zeros((16,), jnp.float32)

    def zbody(i, _):
        acc_o[pl.ds(i * 16, 16)] = zeros
        acc_i[pl.ds(i * 16, 16)] = zeros
        return 0

    lax.fori_loop(0, NPAD // 16, zbody, 0, unroll=8)

    ones = jnp.ones((16,), jnp.float32)
    tail_mask = lax.iota(jnp.int32, 16) >= 8
    base = wid * EPW

    for arr, acc in ((src_hbm, acc_o), (dst_hbm, acc_i)):
        def seg_body(s, _):
            pltpu.sync_copy(arr.at[pl.ds(base + s * SEG, SEG)], seg_v)

            def vbody(i, _):
                iv = seg_v[pl.ds(i * 16, 16)]
                plsc.addupdate_scatter(acc, [iv], ones)
                return 0

            lax.fori_loop(0, SEG // 16, vbody, 0, unroll=8)
            tv = seg_v[pl.ds(SEG - 16, 16)]
            plsc.addupdate_scatter(acc, [tv], ones, mask=tail_mask)
            return 0

        lax.fori_loop(0, EPW // SEG, seg_body, 0)

    pltpu.sync_copy(acc_o, dego_hbm.at[wid])
    pltpu.sync_copy(acc_i, degi_hbm.at[wid])


def _norms_body(po_ref, pi_ref, out_ref):
    do = jnp.sum(po_ref[...], axis=0, keepdims=True)
    di = jnp.sum(pi_ref[...], axis=0, keepdims=True)
    no = jnp.where(do > 0, lax.rsqrt(do), 0.0)
    ni = jnp.where(di > 0, lax.rsqrt(di), 0.0)
    pad = jnp.zeros((6,) + no.shape[1:], jnp.float32)
    out_ref[...] = jnp.concatenate([no, ni, pad], axis=0)


def _norms(dego_p, degi_p):
    blk = 512
    return pl.pallas_call(
        _norms_body,
        grid=(NPAD // blk,),
        in_specs=[
            pl.BlockSpec((NW, blk), lambda i: (0, i)),
            pl.BlockSpec((NW, blk), lambda i: (0, i)),
        ],
        out_specs=pl.BlockSpec((8, blk), lambda i: (0, i)),
        out_shape=jax.ShapeDtypeStruct((8, NPAD), jnp.float32),
    )(dego_p, degi_p)


def _mm1_body(ns_ref, x_ref, w_ref, o_ref):
    h = lax.dot(x_ref[...], w_ref[...], preferred_element_type=jnp.float32)
    h = h * jnp.transpose(ns_ref[0:1, :])
    for p in range(4):
        o_ref[p] = h[:, 32 * p:32 * (p + 1)]


def _mm1(norms, features, W1):
    blk = 256
    k = features.shape[1]
    return pl.pallas_call(
        _mm1_body,
        grid=(NPAD // blk,),
        in_specs=[
            pl.BlockSpec((8, blk), lambda i: (0, i)),
            pl.BlockSpec((blk, k), lambda i: (i, 0)),
            pl.BlockSpec((k, HID), lambda i: (0, 0)),
        ],
        out_specs=pl.BlockSpec((4, blk, 32), lambda i: (0, i, 0)),
        out_shape=jax.ShapeDtypeStruct((4, NPAD, 32), jnp.float32),
    )(norms, features, W1)


@functools.partial(
    pl.kernel,
    out_type=jax.ShapeDtypeStruct((4 * NPAD, 32), jnp.float32),
    mesh=_sc_mesh,
    scratch_types=[
        pltpu.VMEM((BATCH,), jnp.int32),
        pltpu.VMEM((BATCH,), jnp.int32),
        pltpu.VMEM((BATCH, 32), jnp.float32),
        pltpu.VMEM_SHARED((NPAD, 32), jnp.float32),
        pltpu.VMEM((BATCH, 32), jnp.float32),
    ],
    compiler_params=pltpu.CompilerParams(needs_layout_passes=False,
                                         use_tc_tiling_on_sc=False),
)
def _agg1_kernel(src_hbm, dst_hbm, h4r_hbm, zeros_hbm, agg4_hbm,
                 gidx_v, didx_v, stage_v, acc_sh, zero_v):
    cid = lax.axis_index("c")
    sid = lax.axis_index("s")
    nb = jnp.where(sid < NBATCH % 16, NBATCH // 16 + 1, NBATCH // 16)

    pltpu.sync_copy(zeros_hbm, zero_v)

    for pp in range(2):
        p = cid * 2 + pp
        for t in range(STRIPE // BATCH):
            pltpu.sync_copy(zero_v, acc_sh.at[pl.ds(sid * STRIPE + t * BATCH, BATCH), :])
        pltpu.sync_copy(zero_v.at[pl.ds(0, STRIPE % BATCH), :],
                        acc_sh.at[pl.ds(sid * STRIPE + (STRIPE // BATCH) * BATCH,
                                        STRIPE % BATCH), :])
        plsc.subcore_barrier()

        prow = p * NPAD

        def batch_body(t, _):
            ebase = (sid + 16 * t) * BATCH
            pltpu.sync_copy(src_hbm.at[pl.ds(ebase, BATCH)], gidx_v)
            pltpu.sync_copy(dst_hbm.at[pl.ds(ebase, BATCH)], didx_v)

            def fix_body(j, _):
                gidx_v[pl.ds(j * 16, 16)] = gidx_v[pl.ds(j * 16, 16)] + prow
                return 0

            lax.fori_loop(0, BATCH // 16, fix_body, 0, unroll=8)
            pltpu.sync_copy(h4r_hbm.at[gidx_v], stage_v)
            pltpu.sync_copy(stage_v, acc_sh.at[didx_v], add=True)
            return 0

        lax.fori_loop(0, nb, batch_body, 0)
        plsc.subcore_barrier()

        pltpu.sync_copy(
            acc_sh.at[pl.ds(sid * STRIPE, STRIPE), :],
            agg4_hbm.at[pl.ds(prow + sid * STRIPE, STRIPE), :])
        plsc.subcore_barrier()


def _mm2_body(ns_ref, a_ref, b1_ref, w_ref, o_ref):
    agg = jnp.concatenate([a_ref[0], a_ref[1], a_ref[2], a_ref[3]], axis=1)
    nd = jnp.transpose(ns_ref[1:2, :])
    ns = jnp.transpose(ns_ref[0:1, :])
    x1 = jax.nn.relu(agg * nd + b1_ref[...])
    x1 = x1 * ns
    o_ref[...] = lax.dot(x1, w_ref[...], preferred_element_type=jnp.float32)


def _mm2(norms, agg4, b1r, W2p):
    blk = 256
    return pl.pallas_call(
        _mm2_body,
        grid=(NPAD // blk,),
        in_specs=[
            pl.BlockSpec((8, blk), lambda i: (0, i)),
            pl.BlockSpec((4, blk, 32), lambda i: (0, i, 0)),
            pl.BlockSpec((1, HID), lambda i: (0, 0)),
            pl.BlockSpec((HID, 16), lambda i: (0, 0)),
        ],
        out_specs=pl.BlockSpec((blk, 16), lambda i: (i, 0)),
        out_shape=jax.ShapeDtypeStruct((NPAD, 16), jnp.float32),
    )(norms, agg4, b1r, W2p)


@functools.partial(
    pl.kernel,
    out_type=jax.ShapeDtypeStruct((2 * NPAD, 16), jnp.float32),
    mesh=_sc_mesh,
    scratch_types=[
        pltpu.VMEM((BATCH,), jnp.int32),
        pltpu.VMEM((BATCH,), jnp.int32),
        pltpu.VMEM((BATCH, 16), jnp.float32),
        pltpu.VMEM_SHARED((NPAD, 16), jnp.float32),
        pltpu.VMEM((BATCH, 16), jnp.float32),
    ],
    compiler_params=pltpu.CompilerParams(needs_layout_passes=False,
                                         use_tc_tiling_on_sc=False),
)
def _agg2_kernel(src_hbm, dst_hbm, h2_hbm, zeros_hbm, part_hbm,
                 gidx_v, didx_v, stage_v, acc_sh, zero_v):
    cid = lax.axis_index("c")
    sid = lax.axis_index("s")
    wid = sid * 2 + cid
    nb = jnp.where(wid < NBATCH % NW, NBATCH // NW + 1, NBATCH // NW)

    pltpu.sync_copy(zeros_hbm, zero_v)
    for t in range(STRIPE // BATCH):
        pltpu.sync_copy(zero_v, acc_sh.at[pl.ds(sid * STRIPE + t * BATCH, BATCH), :])
    pltpu.sync_copy(zero_v.at[pl.ds(0, STRIPE % BATCH), :],
                    acc_sh.at[pl.ds(sid * STRIPE + (STRIPE // BATCH) * BATCH,
                                    STRIPE % BATCH), :])
    plsc.subcore_barrier()

    def batch_body(t, _):
        ebase = (wid + NW * t) * BATCH
        pltpu.sync_copy(src_hbm.at[pl.ds(ebase, BATCH)], gidx_v)
        pltpu.sync_copy(dst_hbm.at[pl.ds(ebase, BATCH)], didx_v)
        pltpu.sync_copy(h2_hbm.at[gidx_v], stage_v)
        pltpu.sync_copy(stage_v, acc_sh.at[didx_v], add=True)
        return 0

    lax.fori_loop(0, nb, batch_body, 0)
    plsc.subcore_barrier()

    pltpu.sync_copy(
        acc_sh.at[pl.ds(sid * STRIPE, STRIPE), :],
        part_hbm.at[pl.ds(cid * NPAD + sid * STRIPE, STRIPE), :])


def _final_body(ns_ref, p_ref, b2_ref, o_ref):
    s = p_ref[0] + p_ref[1]
    nd = jnp.transpose(ns_ref[1:2, :])
    o_ref[...] = s * nd + b2_ref[...]


def _final(norms, parts, b2r):
    blk = 512
    return pl.pallas_call(
        _final_body,
        grid=(NPAD // blk,),
        in_specs=[
            pl.BlockSpec((8, blk), lambda i: (0, i)),
            pl.BlockSpec((2, blk, 16), lambda i: (0, i, 0)),
            pl.BlockSpec((1, 16), lambda i: (0, 0)),
        ],
        out_specs=pl.BlockSpec((blk, 16), lambda i: (i, 0)),
        out_shape=jax.ShapeDtypeStruct((NPAD, 16), jnp.float32),
    )(norms, parts, b2r)


def kernel(features, edge_index, W1, b1, W2, b2):
    src = edge_index[0].astype(jnp.int32)
    dst = edge_index[1].astype(jnp.int32)

    dego_p, degi_p = _deg_kernel(src, dst)
    norms = _norms(dego_p, degi_p)

    h4 = _mm1(norms, features, W1)
    h4r = h4.reshape(4 * NPAD, 32)

    zeros32 = jnp.zeros((BATCH, 32), jnp.float32)
    agg4 = _agg1_kernel(src, dst, h4r, zeros32).reshape(4, NPAD, 32)

    b1r = b1.reshape(1, HID)
    W2p = jnp.zeros((HID, 16), jnp.float32).at[:, :OUT].set(W2)
    h2 = _mm2(norms, agg4, b1r, W2p)

    zeros16 = jnp.zeros((BATCH, 16), jnp.float32)
    parts = _agg2_kernel(src, dst, h2, zeros16).reshape(2, NPAD, 16)

    b2r = jnp.zeros((1, 16), jnp.float32).at[0, :OUT].set(b2)
    out = _final(norms, parts, b2r)
    return out[:N_NODES, :OUT]


# speedup vs baseline: 6.3642x; 1.8212x over previous
import functools

import jax
import jax.numpy as jnp
from jax import lax
from jax.experimental import pallas as pl
from jax.experimental.pallas import tpu as pltpu
from jax.experimental.pallas import tpu_sc as plsc

N_NODES = 50000
N_EDGES = 800000
IN_FEATS = 1433
HID = 128
OUT = 7

NPAD = 50176
NW = 32
EPW = N_EDGES // NW
SEG = 5000
BATCH = 128
STRIPE = NPAD // 16
EROWS = 6400
E_PAD = EROWS * BATCH
SEGB = 8
NBUF = 4
ROWS1 = EROWS // 16
ROWS2 = EROWS // NW

_sc_mesh = plsc.VectorSubcoreMesh(core_axis_name="c", subcore_axis_name="s")


@functools.partial(
    pl.kernel,
    out_type=[
        jax.ShapeDtypeStruct((NW, NPAD), jnp.float32),
        jax.ShapeDtypeStruct((NW, NPAD), jnp.float32),
    ],
    mesh=_sc_mesh,
    scratch_types=[
        pltpu.VMEM((NPAD,), jnp.float32),
        pltpu.VMEM((NPAD,), jnp.float32),
        pltpu.VMEM((SEG,), jnp.int32),
    ],
    compiler_params=pltpu.CompilerParams(needs_layout_passes=False),
)
def _deg_kernel(src_hbm, dst_hbm, dego_hbm, degi_hbm, acc_o, acc_i, seg_v):
    cid = lax.axis_index("c")
    sid = lax.axis_index("s")
    wid = sid * 2 + cid

    zeros = jnp.zeros((16,), jnp.float32)

    def zbody(i, _):
        acc_o[pl.ds(i * 16, 16)] = zeros
        acc_i[pl.ds(i * 16, 16)] = zeros
        return 0

    lax.fori_loop(0, NPAD // 16, zbody, 0, unroll=8)

    ones = jnp.ones((16,), jnp.float32)
    tail_mask = lax.iota(jnp.int32, 16) >= 8
    base = wid * EPW

    for arr, acc in ((src_hbm, acc_o), (dst_hbm, acc_i)):
        def seg_body(s, _):
            pltpu.sync_copy(arr.at[pl.ds(base + s * SEG, SEG)], seg_v)

            def vbody(i, _):
                iv = seg_v[pl.ds(i * 16, 16)]
                plsc.addupdate_scatter(acc, [iv], ones)
                return 0

            lax.fori_loop(0, SEG // 16, vbody, 0, unroll=8)
            tv = seg_v[pl.ds(SEG - 16, 16)]
            plsc.addupdate_scatter(acc, [tv], ones, mask=tail_mask)
            return 0

        lax.fori_loop(0, EPW // SEG, seg_body, 0)

    pltpu.sync_copy(acc_o, dego_hbm.at[wid])
    pltpu.sync_copy(acc_i, degi_hbm.at[wid])


def _norms_body(po_ref, pi_ref, out_ref):
    do = jnp.sum(po_ref[...], axis=0, keepdims=True)
    di = jnp.sum(pi_ref[...], axis=0, keepdims=True)
    no = jnp.where(do > 0, lax.rsqrt(do), 0.0)
    ni = jnp.where(di > 0, lax.rsqrt(di), 0.0)
    pad = jnp.zeros((6,) + no.shape[1:], jnp.float32)
    out_ref[...] = jnp.concatenate([no, ni, pad], axis=0)


def _norms(dego_p, degi_p):
    blk = 512
    return pl.pallas_call(
        _norms_body,
        grid=(NPAD // blk,),
        in_specs=[
            pl.BlockSpec((NW, blk), lambda i: (0, i)),
            pl.BlockSpec((NW, blk), lambda i: (0, i)),
        ],
        out_specs=pl.BlockSpec((8, blk), lambda i: (0, i)),
        out_shape=jax.ShapeDtypeStruct((8, NPAD), jnp.float32),
    )(dego_p, degi_p)


def _mm1_body(ns_ref, x_ref, w_ref, o_ref):
    h = lax.dot(x_ref[...], w_ref[...], preferred_element_type=jnp.float32)
    h = h * jnp.transpose(ns_ref[0:1, :])
    for p in range(4):
        o_ref[p] = h[:, 32 * p:32 * (p + 1)]


def _mm1(norms, features, W1):
    blk = 256
    k = features.shape[1]
    return pl.pallas_call(
        _mm1_body,
        grid=(NPAD // blk,),
        in_specs=[
            pl.BlockSpec((8, blk), lambda i: (0, i)),
            pl.BlockSpec((blk, k), lambda i: (i, 0)),
            pl.BlockSpec((k, HID), lambda i: (0, 0)),
        ],
        out_specs=pl.BlockSpec((4, blk, 32), lambda i: (0, i, 0)),
        out_shape=jax.ShapeDtypeStruct((4, NPAD, 32), jnp.float32),
    )(norms, features, W1)


@functools.partial(
    pl.kernel,
    out_type=jax.ShapeDtypeStruct((4 * NPAD, 32), jnp.float32),
    mesh=_sc_mesh,
    scratch_types=(
        [
            pltpu.VMEM((SEGB * BATCH,), jnp.int32),
            pltpu.VMEM((SEGB, BATCH), jnp.int32),
            pltpu.VMEM_SHARED((NPAD, 32), jnp.float32),
            pltpu.VMEM((BATCH, 32), jnp.float32),
        ]
        + [pltpu.VMEM((BATCH, 32), jnp.float32) for _ in range(NBUF)]
        + [pltpu.SemaphoreType.DMA for _ in range(NBUF)]
    ),
    compiler_params=pltpu.CompilerParams(needs_layout_passes=False,
                                         use_tc_tiling_on_sc=False),
)
def _agg1_kernel(src_hbm, dst2d_hbm, h4r_hbm, zeros_hbm, agg4_hbm,
                 gidx_v, didx_v, acc_sh, zero_v, *stage_sems):
    stages = stage_sems[:NBUF]
    sems = stage_sems[NBUF:]
    cid = lax.axis_index("c")
    sid = lax.axis_index("s")

    pltpu.sync_copy(zeros_hbm, zero_v)

    for pp in range(2):
        p = cid * 2 + pp
        for t in range(STRIPE // BATCH):
            pltpu.sync_copy(zero_v, acc_sh.at[pl.ds(sid * STRIPE + t * BATCH, BATCH), :])
        pltpu.sync_copy(zero_v.at[pl.ds(0, STRIPE % BATCH), :],
                        acc_sh.at[pl.ds(sid * STRIPE + (STRIPE // BATCH) * BATCH,
                                        STRIPE % BATCH), :])
        plsc.subcore_barrier()

        prow = p * NPAD

        def seg_body(s, _):
            r0 = sid * ROWS1 + s * SEGB
            pltpu.sync_copy(src_hbm.at[pl.ds(r0 * BATCH, SEGB * BATCH)], gidx_v)
            pltpu.sync_copy(dst2d_hbm.at[pl.ds(r0, SEGB), :], didx_v)

            def fix_body(j, _):
                gidx_v[pl.ds(j * 16, 16)] = gidx_v[pl.ds(j * 16, 16)] + prow
                return 0

            lax.fori_loop(0, SEGB * BATCH // 16, fix_body, 0, unroll=8)

            descs = []
            for b in range(NBUF):
                descs.append(pltpu.async_copy(
                    h4r_hbm.at[gidx_v.at[pl.ds(b * BATCH, BATCH)]],
                    stages[b], sems[b]))
            for b in range(SEGB):
                descs[b].wait()
                pltpu.sync_copy(stages[b % NBUF], acc_sh.at[didx_v.at[b]],
                                add=True)
                if b + NBUF < SEGB:
                    descs.append(pltpu.async_copy(
                        h4r_hbm.at[gidx_v.at[pl.ds((b + NBUF) * BATCH, BATCH)]],
                        stages[b % NBUF], sems[b % NBUF]))
            return 0

        lax.fori_loop(0, ROWS1 // SEGB, seg_body, 0)
        plsc.subcore_barrier()

        pltpu.sync_copy(
            acc_sh.at[pl.ds(sid * STRIPE, STRIPE), :],
            agg4_hbm.at[pl.ds(prow + sid * STRIPE, STRIPE), :])
        plsc.subcore_barrier()


def _mm2_body(ns_ref, a_ref, b1_ref, w_ref, o_ref):
    agg = jnp.concatenate([a_ref[0], a_ref[1], a_ref[2], a_ref[3]], axis=1)
    nd = jnp.transpose(ns_ref[1:2, :])
    ns = jnp.transpose(ns_ref[0:1, :])
    x1 = jax.nn.relu(agg * nd + b1_ref[...])
    x1 = x1 * ns
    o_ref[...] = lax.dot(x1, w_ref[...], preferred_element_type=jnp.float32)


def _mm2(norms, agg4, b1r, W2p):
    blk = 256
    return pl.pallas_call(
        _mm2_body,
        grid=(NPAD // blk,),
        in_specs=[
            pl.BlockSpec((8, blk), lambda i: (0, i)),
            pl.BlockSpec((4, blk, 32), lambda i: (0, i, 0)),
            pl.BlockSpec((1, HID), lambda i: (0, 0)),
            pl.BlockSpec((HID, 16), lambda i: (0, 0)),
        ],
        out_specs=pl.BlockSpec((blk, 16), lambda i: (i, 0)),
        out_shape=jax.ShapeDtypeStruct((NPAD, 16), jnp.float32),
    )(norms, agg4, b1r, W2p)


@functools.partial(
    pl.kernel,
    out_type=jax.ShapeDtypeStruct((2 * NPAD, 16), jnp.float32),
    mesh=_sc_mesh,
    scratch_types=(
        [
            pltpu.VMEM((SEGB, BATCH), jnp.int32),
            pltpu.VMEM((SEGB, BATCH), jnp.int32),
            pltpu.VMEM_SHARED((NPAD, 16), jnp.float32),
            pltpu.VMEM((BATCH, 16), jnp.float32),
        ]
        + [pltpu.VMEM((BATCH, 16), jnp.float32) for _ in range(NBUF)]
        + [pltpu.SemaphoreType.DMA for _ in range(NBUF)]
    ),
    compiler_params=pltpu.CompilerParams(needs_layout_passes=False,
                                         use_tc_tiling_on_sc=False),
)
def _agg2_kernel(src2d_hbm, dst2d_hbm, h2_hbm, zeros_hbm, part_hbm,
                 gidx_v, didx_v, acc_sh, zero_v, *stage_sems):
    stages = stage_sems[:NBUF]
    sems = stage_sems[NBUF:]
    cid = lax.axis_index("c")
    sid = lax.axis_index("s")
    wid = sid * 2 + cid

    pltpu.sync_copy(zeros_hbm, zero_v)
    for t in range(STRIPE // BATCH):
        pltpu.sync_copy(zero_v, acc_sh.at[pl.ds(sid * STRIPE + t * BATCH, BATCH), :])
    pltpu.sync_copy(zero_v.at[pl.ds(0, STRIPE % BATCH), :],
                    acc_sh.at[pl.ds(sid * STRIPE + (STRIPE // BATCH) * BATCH,
                                    STRIPE % BATCH), :])
    plsc.subcore_barrier()

    def seg_body(s, _):
        r0 = wid * ROWS2 + s * SEGB
        pltpu.sync_copy(src2d_hbm.at[pl.ds(r0, SEGB), :], gidx_v)
        pltpu.sync_copy(dst2d_hbm.at[pl.ds(r0, SEGB), :], didx_v)

        descs = []
        for b in range(NBUF):
            descs.append(pltpu.async_copy(
                h2_hbm.at[gidx_v.at[b]], stages[b], sems[b]))
        for b in range(SEGB):
            descs[b].wait()
            pltpu.sync_copy(stages[b % NBUF], acc_sh.at[didx_v.at[b]],
                            add=True)
            if b + NBUF < SEGB:
                descs.append(pltpu.async_copy(
                    h2_hbm.at[gidx_v.at[b + NBUF]],
                    stages[b % NBUF], sems[b % NBUF]))
        return 0

    lax.fori_loop(0, ROWS2 // SEGB, seg_body, 0)
    plsc.subcore_barrier()

    pltpu.sync_copy(
        acc_sh.at[pl.ds(sid * STRIPE, STRIPE), :],
        part_hbm.at[pl.ds(cid * NPAD + sid * STRIPE, STRIPE), :])


def _final_body(ns_ref, p_ref, b2_ref, o_ref):
    s = p_ref[0] + p_ref[1]
    nd = jnp.transpose(ns_ref[1:2, :])
    o_ref[...] = s * nd + b2_ref[...]


def _final(norms, parts, b2r):
    blk = 512
    return pl.pallas_call(
        _final_body,
        grid=(NPAD // blk,),
        in_specs=[
            pl.BlockSpec((8, blk), lambda i: (0, i)),
            pl.BlockSpec((2, blk, 16), lambda i: (0, i, 0)),
            pl.BlockSpec((1, 16), lambda i: (0, 0)),
        ],
        out_specs=pl.BlockSpec((blk, 16), lambda i: (i, 0)),
        out_shape=jax.ShapeDtypeStruct((NPAD, 16), jnp.float32),
    )(norms, parts, b2r)


def kernel(features, edge_index, W1, b1, W2, b2):
    src = edge_index[0].astype(jnp.int32)
    dst = edge_index[1].astype(jnp.int32)

    npad_e = E_PAD - N_EDGES
    ar = jnp.arange(npad_e, dtype=jnp.int32)
    src_pad = jnp.concatenate([src, ar % N_NODES])
    dst_pad = jnp.concatenate([dst, NPAD - 16 + (ar % 16)])
    src2d = src_pad.reshape(EROWS, BATCH)
    dst2d = dst_pad.reshape(EROWS, BATCH)

    dego_p, degi_p = _deg_kernel(src, dst)
    norms = _norms(dego_p, degi_p)

    h4 = _mm1(norms, features, W1)
    h4r = h4.reshape(4 * NPAD, 32)

    zeros32 = jnp.zeros((BATCH, 32), jnp.float32)
    agg4 = _agg1_kernel(src_pad, dst2d, h4r, zeros32).reshape(4, NPAD, 32)

    b1r = b1.reshape(1, HID)
    W2p = jnp.zeros((HID, 16), jnp.float32).at[:, :OUT].set(W2)
    h2 = _mm2(norms, agg4, b1r, W2p)

    zeros16 = jnp.zeros((BATCH, 16), jnp.float32)
    parts = _agg2_kernel(src2d, dst2d, h2, zeros16).reshape(2, NPAD, 16)

    b2r = jnp.zeros((1, 16), jnp.float32).at[0, :OUT].set(b2)
    out = _final(norms, parts, b2r)
    return out[:N_NODES, :OUT]
